# Initial kernel scaffold; baseline (speedup 1.0000x reference)
import jax
import jax.numpy as jnp
from jax.experimental import pallas as pl


def kernel(byte_ids, charge_table, position_table, mass_table):
    raise NotImplementedError("write your pallas kernel here")


# speedup vs baseline: 1.5878x; 1.5878x over previous
import functools
import math

import jax
import jax.numpy as jnp
import numpy as np
from jax import lax
from jax.experimental import pallas as pl
from jax.experimental.pallas import tpu as pltpu
from jax.experimental.pallas import tpu_sc as plsc

D_MODEL = 1024
POS_DIM = 16
B, L = 4, 4096
N_TOK = B * L
NC, NS, LANES = 2, 16, 16
NW = NC * NS
TOK_PER_W = N_TOK // NW
SLAB = 32
N_SLAB = TOK_PER_W // SLAB
W_PER_BATCH = L // TOK_PER_W


def _pe_table():
    position = np.arange(L)[:, None].astype(np.float32)
    div_term = np.exp(
        np.arange(0, D_MODEL, 2).astype(np.float32) * (-math.log(10000.0) / D_MODEL)
    )
    pe = np.zeros((L, D_MODEL), dtype=np.float32)
    pe[:, 0::2] = np.sin(position * div_term)
    pe[:, 1::2] = np.cos(position * div_term)
    return jnp.asarray(pe)


def _sc_body(ids_hbm, charge_hbm, pos_hbm, mass_hbm, pe_hbm,
             charge_out, pos_out, mass_out,
             idx_v, rows_v, pe_v, ptab_v, pos_v, mtab_v, mass_v, sem_g, sem_p):
    wid = lax.axis_index("s") * NC + lax.axis_index("c")
    base = wid * TOK_PER_W
    l_base = (wid % W_PER_BATCH) * TOK_PER_W

    pltpu.sync_copy(ids_hbm.at[pl.ds(base, TOK_PER_W)], idx_v)
    pltpu.sync_copy(pos_hbm, ptab_v)
    pltpu.sync_copy(mass_hbm, mtab_v)

    lane_iota = lax.iota(jnp.int32, LANES)

    def _pos_step(j, carry):
        ids16 = idx_v[pl.ds(j * LANES, LANES)]
        flat_base = ids16 * POS_DIM
        out_base = j * (LANES * POS_DIM) + lane_iota * POS_DIM
        for c in range(POS_DIM):
            vals = plsc.load_gather(ptab_v, [flat_base + c])
            plsc.store_scatter(pos_v, [out_base + c], vals)
        return carry

    lax.fori_loop(0, TOK_PER_W // LANES, _pos_step, 0)
    pltpu.sync_copy(pos_v, pos_out.at[pl.ds(base * POS_DIM, TOK_PER_W * POS_DIM)])

    def _mass_step(j, carry):
        ids16 = idx_v[pl.ds(j * LANES, LANES)]
        x = plsc.load_gather(mtab_v, [ids16])
        mass_v[pl.ds(j * LANES, LANES)] = 1.0 / (1.0 + jnp.exp(-x))
        return carry

    lax.fori_loop(0, TOK_PER_W // LANES, _mass_step, 0)
    pltpu.sync_copy(mass_v, mass_out.at[pl.ds(base, TOK_PER_W)])

    def _slab(s, carry):
        cp_rows = pltpu.async_copy(
            charge_hbm.at[idx_v.at[pl.ds(s * SLAB, SLAB)]], rows_v, sem_g)
        cp_pe = pltpu.async_copy(
            pe_hbm.at[pl.ds(l_base + s * SLAB, SLAB)], pe_v, sem_p)
        cp_rows.wait()
        cp_pe.wait()

        def _row(i, c):
            for k in range(D_MODEL // LANES):
                sl = pl.ds(k * LANES, LANES)
                pe_v[i, sl] = pe_v[i, sl] + rows_v[i, sl]
            return c

        lax.fori_loop(0, SLAB, _row, 0)
        pltpu.sync_copy(pe_v, charge_out.at[pl.ds(base + s * SLAB, SLAB)])
        return carry

    lax.fori_loop(0, N_SLAB, _slab, 0)


@functools.partial(
    pl.kernel,
    out_type=[
        jax.ShapeDtypeStruct((N_TOK, D_MODEL), jnp.float32),
        jax.ShapeDtypeStruct((N_TOK * POS_DIM,), jnp.float32),
        jax.ShapeDtypeStruct((N_TOK,), jnp.float32),
    ],
    scratch_types=[
        pltpu.VMEM((TOK_PER_W,), jnp.int32),
        pltpu.VMEM((SLAB, D_MODEL), jnp.float32),
        pltpu.VMEM((SLAB, D_MODEL), jnp.float32),
        pltpu.VMEM((256 * POS_DIM,), jnp.float32),
        pltpu.VMEM((TOK_PER_W * POS_DIM,), jnp.float32),
        pltpu.VMEM((256,), jnp.float32),
        pltpu.VMEM((TOK_PER_W,), jnp.float32),
        pltpu.SemaphoreType.DMA,
        pltpu.SemaphoreType.DMA,
    ],
    mesh=plsc.VectorSubcoreMesh(core_axis_name="c", subcore_axis_name="s"),
    compiler_params=pltpu.CompilerParams(needs_layout_passes=False),
)
def _byte_to_particle_sc(*args):
    _sc_body(*args)


def kernel(byte_ids, charge_table, position_table, mass_table):
    assert byte_ids.shape == (B, L)
    ids_flat = byte_ids.reshape(N_TOK).astype(jnp.int32)
    pe = _pe_table()
    charge_f, pos_f, mass_f = _byte_to_particle_sc(
        ids_flat, charge_table, position_table.reshape(256 * POS_DIM),
        mass_table.reshape(256), pe)
    return (
        charge_f.reshape(B, L, D_MODEL),
        pos_f.reshape(B, L, POS_DIM),
        mass_f.reshape(B, L, 1),
    )


# speedup vs baseline: 2.2966x; 1.4464x over previous
import functools
import math

import jax
import jax.numpy as jnp
import numpy as np
from jax import lax
from jax.experimental import pallas as pl
from jax.experimental.pallas import tpu as pltpu
from jax.experimental.pallas import tpu_sc as plsc

D_MODEL = 1024
POS_DIM = 16
B, L = 4, 4096
N_TOK = B * L
NC, NS, LANES = 2, 16, 16
NW = NC * NS
TOK_PER_W = N_TOK // NW
L_PER_W = L // NW
SLAB_L = 16
N_SLAB = L_PER_W // SLAB_L
N_STEP = N_SLAB * B


def _pe_table():
    position = np.arange(L)[:, None].astype(np.float32)
    div_term = np.exp(
        np.arange(0, D_MODEL, 2).astype(np.float32) * (-math.log(10000.0) / D_MODEL)
    )
    pe = np.zeros((L, D_MODEL), dtype=np.float32)
    pe[:, 0::2] = np.sin(position * div_term)
    pe[:, 1::2] = np.cos(position * div_term)
    return jnp.asarray(pe)


def _sc_body(ids_hbm, charge_hbm, pos_hbm, mass_hbm, pe_hbm,
             charge_out, pos_out, mass_out,
             idx_v, rows0, rows1, rows2, pe0, pe1, ptab_v, pos_v, mtab_v,
             mass_v, sem_g, sem_p, sem_s):
    wid = lax.axis_index("s") * NC + lax.axis_index("c")
    w128 = wid * L_PER_W
    rows = (rows0, rows1, rows2)
    pes = (pe0, pe1)

    for b in range(B):
        pltpu.sync_copy(ids_hbm.at[pl.ds(b * L + w128, L_PER_W)],
                        idx_v.at[pl.ds(b * L_PER_W, L_PER_W)])

    def g_issue(k):
        s, b = divmod(k, B)
        pltpu.async_copy(
            charge_hbm.at[idx_v.at[pl.ds(b * L_PER_W + s * SLAB_L, SLAB_L)]],
            rows[k % 3], sem_g)

    def g_drain(k):
        pltpu.make_async_copy(
            charge_hbm.at[idx_v.at[pl.ds(0, SLAB_L)]], rows[k % 3], sem_g).wait()

    def s_issue(k):
        s, b = divmod(k, B)
        pltpu.async_copy(
            rows[k % 3],
            charge_out.at[pl.ds(b * L + w128 + s * SLAB_L, SLAB_L)], sem_s)

    def s_drain():
        pltpu.make_async_copy(rows0, charge_out.at[pl.ds(0, SLAB_L)], sem_s).wait()

    def pe_issue(s):
        pltpu.async_copy(pe_hbm.at[pl.ds(w128 + s * SLAB_L, SLAB_L)],
                         pes[s % 2], sem_p)

    def pe_drain(s):
        pltpu.make_async_copy(pe_hbm.at[pl.ds(0, SLAB_L)], pes[s % 2], sem_p).wait()

    pe_issue(0)
    pe_issue(1)
    g_issue(0)

    pltpu.sync_copy(pos_hbm, ptab_v)
    pltpu.sync_copy(mass_hbm, mtab_v)
    lane_iota = lax.iota(jnp.int32, LANES)

    def _pos_step(j, carry):
        ids16 = idx_v[pl.ds(j * LANES, LANES)]
        flat_base = ids16 * POS_DIM
        out_base = j * (LANES * POS_DIM) + lane_iota * POS_DIM
        for c in range(POS_DIM):
            vals = plsc.load_gather(ptab_v, [flat_base + c])
            plsc.store_scatter(pos_v, [out_base + c], vals)
        return carry

    lax.fori_loop(0, TOK_PER_W // LANES, _pos_step, 0)
    for b in range(B):
        pltpu.sync_copy(
            pos_v.at[pl.ds(b * L_PER_W * POS_DIM, L_PER_W * POS_DIM)],
            pos_out.at[pl.ds((b * L + w128) * POS_DIM, L_PER_W * POS_DIM)])

    def _mass_step(j, carry):
        ids16 = idx_v[pl.ds(j * LANES, LANES)]
        x = plsc.load_gather(mtab_v, [ids16])
        mass_v[pl.ds(j * LANES, LANES)] = 1.0 / (1.0 + jnp.exp(-x))
        return carry

    lax.fori_loop(0, TOK_PER_W // LANES, _mass_step, 0)
    for b in range(B):
        pltpu.sync_copy(mass_v.at[pl.ds(b * L_PER_W, L_PER_W)],
                        mass_out.at[pl.ds(b * L + w128, L_PER_W)])

    for k in range(N_STEP):
        s, b = divmod(k, B)
        if k >= 2:
            s_drain()
        if k + 1 < N_STEP:
            g_issue(k + 1)
        g_drain(k)
        if b == 0:
            pe_drain(s)

        rk, pk = rows[k % 3], pes[s % 2]

        def _row(i, carry, rk=rk, pk=pk):
            for c in range(D_MODEL // LANES):
                sl = pl.ds(c * LANES, LANES)
                plsc.addupdate(rk.at[i, sl], pk[i, sl])
            return carry

        lax.fori_loop(0, SLAB_L, _row, 0)
        s_issue(k)
        if b == B - 1 and s + 2 < N_SLAB:
            pe_issue(s + 2)

    s_drain()
    s_drain()


@functools.partial(
    pl.kernel,
    out_type=[
        jax.ShapeDtypeStruct((N_TOK, D_MODEL), jnp.float32),
        jax.ShapeDtypeStruct((N_TOK * POS_DIM,), jnp.float32),
        jax.ShapeDtypeStruct((N_TOK,), jnp.float32),
    ],
    scratch_types=[
        pltpu.VMEM((TOK_PER_W,), jnp.int32),
        pltpu.VMEM((SLAB_L, D_MODEL), jnp.float32),
        pltpu.VMEM((SLAB_L, D_MODEL), jnp.float32),
        pltpu.VMEM((SLAB_L, D_MODEL), jnp.float32),
        pltpu.VMEM((SLAB_L, D_MODEL), jnp.float32),
        pltpu.VMEM((SLAB_L, D_MODEL), jnp.float32),
        pltpu.VMEM((256 * POS_DIM,), jnp.float32),
        pltpu.VMEM((TOK_PER_W * POS_DIM,), jnp.float32),
        pltpu.VMEM((256,), jnp.float32),
        pltpu.VMEM((TOK_PER_W,), jnp.float32),
        pltpu.SemaphoreType.DMA,
        pltpu.SemaphoreType.DMA,
        pltpu.SemaphoreType.DMA,
    ],
    mesh=plsc.VectorSubcoreMesh(core_axis_name="c", subcore_axis_name="s"),
    compiler_params=pltpu.CompilerParams(needs_layout_passes=False),
)
def _byte_to_particle_sc(*args):
    _sc_body(*args)


def kernel(byte_ids, charge_table, position_table, mass_table):
    assert byte_ids.shape == (B, L)
    ids_flat = byte_ids.reshape(N_TOK).astype(jnp.int32)
    pe = _pe_table()
    charge_f, pos_f, mass_f = _byte_to_particle_sc(
        ids_flat, charge_table, position_table.reshape(256 * POS_DIM),
        mass_table.reshape(256), pe)
    return (
        charge_f.reshape(B, L, D_MODEL),
        pos_f.reshape(B, L, POS_DIM),
        mass_f.reshape(B, L, 1),
    )


# speedup vs baseline: 3.7899x; 1.6503x over previous
import functools
import math

import jax
import jax.numpy as jnp
import numpy as np
from jax import lax
from jax.experimental import pallas as pl
from jax.experimental.pallas import tpu as pltpu
from jax.experimental.pallas import tpu_sc as plsc

D_MODEL = 1024
POS_DIM = 16
B, L = 4, 4096
N_TOK = B * L
NC, NS, LANES = 2, 16, 16
NW = NC * NS
TOK_PER_W = N_TOK // NW
L_PER_W = L // NW

TC_BLK = 512
N_LBLK = L // TC_BLK


def _pe_table():
    position = np.arange(L)[:, None].astype(np.float32)
    div_term = np.exp(
        np.arange(0, D_MODEL, 2).astype(np.float32) * (-math.log(10000.0) / D_MODEL)
    )
    pe = np.zeros((L, D_MODEL), dtype=np.float32)
    pe[:, 0::2] = np.sin(position * div_term)
    pe[:, 1::2] = np.cos(position * div_term)
    return jnp.asarray(pe)


def _charge_tc_body(ids_ref, tab_ref, pe_ref, out_ref):
    ids = ids_ref[0, 0, :]
    onehot = (ids[:, None] == lax.broadcasted_iota(jnp.int32, (TC_BLK, 256), 1))
    onehot = onehot.astype(jnp.float32)
    rows = jax.lax.dot_general(
        onehot, tab_ref[...],
        dimension_numbers=(((1,), (0,)), ((), ())),
        preferred_element_type=jnp.float32)
    out_ref[...] = rows + pe_ref[...]


def _charge_tc(ids32, charge_table, pe):
    return pl.pallas_call(
        _charge_tc_body,
        grid=(N_LBLK, B),
        in_specs=[
            pl.BlockSpec((1, 1, TC_BLK), lambda l, b: (b * N_LBLK + l, 0, 0)),
            pl.BlockSpec((256, D_MODEL), lambda l, b: (0, 0)),
            pl.BlockSpec((TC_BLK, D_MODEL), lambda l, b: (l, 0)),
        ],
        out_specs=pl.BlockSpec((TC_BLK, D_MODEL), lambda l, b: (b * N_LBLK + l, 0)),
        out_shape=jax.ShapeDtypeStruct((N_TOK, D_MODEL), jnp.float32),
    )(ids32, charge_table, pe)


def _sc_body(ids_hbm, pos_hbm, mass_hbm,
             pos_out, mass_out,
             idx_v, ptab_v, pos_v, mtab_v, mass_v):
    wid = lax.axis_index("s") * NC + lax.axis_index("c")
    w128 = wid * L_PER_W

    for b in range(B):
        pltpu.sync_copy(ids_hbm.at[pl.ds(b * L + w128, L_PER_W)],
                        idx_v.at[pl.ds(b * L_PER_W, L_PER_W)])
    pltpu.sync_copy(pos_hbm, ptab_v)
    pltpu.sync_copy(mass_hbm, mtab_v)
    lane_iota = lax.iota(jnp.int32, LANES)

    def _pos_step(j, carry):
        ids16 = idx_v[pl.ds(j * LANES, LANES)]
        flat_base = ids16 * POS_DIM
        out_base = j * (LANES * POS_DIM) + lane_iota * POS_DIM
        for c in range(POS_DIM):
            vals = plsc.load_gather(ptab_v, [flat_base + c])
            plsc.store_scatter(pos_v, [out_base + c], vals)
        return carry

    lax.fori_loop(0, TOK_PER_W // LANES, _pos_step, 0)
    for b in range(B):
        pltpu.sync_copy(
            pos_v.at[pl.ds(b * L_PER_W * POS_DIM, L_PER_W * POS_DIM)],
            pos_out.at[pl.ds((b * L + w128) * POS_DIM, L_PER_W * POS_DIM)])

    def _mass_step(j, carry):
        ids16 = idx_v[pl.ds(j * LANES, LANES)]
        x = plsc.load_gather(mtab_v, [ids16])
        mass_v[pl.ds(j * LANES, LANES)] = 1.0 / (1.0 + jnp.exp(-x))
        return carry

    lax.fori_loop(0, TOK_PER_W // LANES, _mass_step, 0)
    for b in range(B):
        pltpu.sync_copy(mass_v.at[pl.ds(b * L_PER_W, L_PER_W)],
                        mass_out.at[pl.ds(b * L + w128, L_PER_W)])


@functools.partial(
    pl.kernel,
    out_type=[
        jax.ShapeDtypeStruct((N_TOK * POS_DIM,), jnp.float32),
        jax.ShapeDtypeStruct((N_TOK,), jnp.float32),
    ],
    scratch_types=[
        pltpu.VMEM((TOK_PER_W,), jnp.int32),
        pltpu.VMEM((256 * POS_DIM,), jnp.float32),
        pltpu.VMEM((TOK_PER_W * POS_DIM,), jnp.float32),
        pltpu.VMEM((256,), jnp.float32),
        pltpu.VMEM((TOK_PER_W,), jnp.float32),
    ],
    mesh=plsc.VectorSubcoreMesh(core_axis_name="c", subcore_axis_name="s"),
    compiler_params=pltpu.CompilerParams(needs_layout_passes=False),
)
def _pos_mass_sc(*args):
    _sc_body(*args)


def kernel(byte_ids, charge_table, position_table, mass_table):
    assert byte_ids.shape == (B, L)
    ids_flat = byte_ids.reshape(N_TOK).astype(jnp.int32)
    ids_blk = byte_ids.reshape(NW, 1, TC_BLK).astype(jnp.int32)
    pe = _pe_table()
    charge_f = _charge_tc(ids_blk, charge_table, pe)
    pos_f, mass_f = _pos_mass_sc(
        ids_flat, position_table.reshape(256 * POS_DIM), mass_table.reshape(256))
    return (
        charge_f.reshape(B, L, D_MODEL),
        pos_f.reshape(B, L, POS_DIM),
        mass_f.reshape(B, L, 1),
    )


# speedup vs baseline: 4.2293x; 1.1159x over previous
import functools
import math

import jax
import jax.numpy as jnp
import numpy as np
from jax import lax
from jax.experimental import pallas as pl
from jax.experimental.pallas import tpu as pltpu
from jax.experimental.pallas import tpu_sc as plsc

D_MODEL = 1024
POS_DIM = 16
B, L = 4, 4096
N_TOK = B * L
NC, NS, LANES = 2, 16, 16
NW = NC * NS
TOK_PER_W = N_TOK // NW
L_PER_W = L // NW

TC_BLK = 512
N_LBLK = L // TC_BLK


def _pe_table():
    position = np.arange(L)[:, None].astype(np.float32)
    div_term = np.exp(
        np.arange(0, D_MODEL, 2).astype(np.float32) * (-math.log(10000.0) / D_MODEL)
    )
    pe = np.zeros((L, D_MODEL), dtype=np.float32)
    pe[:, 0::2] = np.sin(position * div_term)
    pe[:, 1::2] = np.cos(position * div_term)
    return jnp.asarray(pe)


def _charge_tc_body(ids_ref, tab_ref, pe_ref, out_ref):
    ids = ids_ref[0, 0, :]
    onehot = (ids[:, None] == lax.broadcasted_iota(jnp.int32, (TC_BLK, 256), 1))
    onehot = onehot.astype(jnp.bfloat16)
    rows = jax.lax.dot_general(
        onehot, tab_ref[...],
        dimension_numbers=(((1,), (0,)), ((), ())),
        preferred_element_type=jnp.float32)
    out_ref[...] = rows + pe_ref[...]


def _charge_tc(ids32, charge_table, pe):
    return pl.pallas_call(
        _charge_tc_body,
        grid=(N_LBLK, B),
        in_specs=[
            pl.BlockSpec((1, 1, TC_BLK), lambda l, b: (b * N_LBLK + l, 0, 0)),
            pl.BlockSpec((256, D_MODEL), lambda l, b: (0, 0)),
            pl.BlockSpec((TC_BLK, D_MODEL), lambda l, b: (l, 0)),
        ],
        out_specs=pl.BlockSpec((TC_BLK, D_MODEL), lambda l, b: (b * N_LBLK + l, 0)),
        out_shape=jax.ShapeDtypeStruct((N_TOK, D_MODEL), jnp.float32),
    )(ids32, charge_table, pe)


def _sc_body(ids_hbm, pos_hbm, mass_hbm,
             pos_out, mass_out,
             idx_v, ptab_v, pos_v, mtab_v, mass_v, sem_o):
    wid = lax.axis_index("s") * NC + lax.axis_index("c")
    w128 = wid * L_PER_W

    for b in range(B):
        pltpu.sync_copy(ids_hbm.at[pl.ds(b * L + w128, L_PER_W)],
                        idx_v.at[pl.ds(b * L_PER_W, L_PER_W)])
    pltpu.sync_copy(pos_hbm, ptab_v)
    pltpu.sync_copy(mass_hbm, mtab_v)

    def _pos_step(j, carry):
        ids16 = idx_v[pl.ds(j * LANES, LANES)]
        flat_base = ids16 * POS_DIM
        stage = (j // 8) * (POS_DIM * L_PER_W) + (j % 8) * LANES
        for c in range(POS_DIM):
            vals = plsc.load_gather(ptab_v, [flat_base + c])
            pos_v[pl.ds(stage + c * L_PER_W, LANES)] = vals
        return carry

    lax.fori_loop(0, TOK_PER_W // LANES, _pos_step, 0)
    for b in range(B):
        cps = []
        for c in range(POS_DIM):
            cps.append(pltpu.async_copy(
                pos_v.at[pl.ds(b * (POS_DIM * L_PER_W) + c * L_PER_W, L_PER_W)],
                pos_out.at[pl.ds((b * POS_DIM + c) * L + w128, L_PER_W)],
                sem_o))
        for cp in cps:
            cp.wait()

    def _mass_step(j, carry):
        ids16 = idx_v[pl.ds(j * LANES, LANES)]
        x = plsc.load_gather(mtab_v, [ids16])
        mass_v[pl.ds(j * LANES, LANES)] = 1.0 / (1.0 + jnp.exp(-x))
        return carry

    lax.fori_loop(0, TOK_PER_W // LANES, _mass_step, 0)
    for b in range(B):
        pltpu.sync_copy(mass_v.at[pl.ds(b * L_PER_W, L_PER_W)],
                        mass_out.at[pl.ds(b * L + w128, L_PER_W)])


@functools.partial(
    pl.kernel,
    out_type=[
        jax.ShapeDtypeStruct((N_TOK * POS_DIM,), jnp.float32),
        jax.ShapeDtypeStruct((N_TOK,), jnp.float32),
    ],
    scratch_types=[
        pltpu.VMEM((TOK_PER_W,), jnp.int32),
        pltpu.VMEM((256 * POS_DIM,), jnp.float32),
        pltpu.VMEM((TOK_PER_W * POS_DIM,), jnp.float32),
        pltpu.VMEM((256,), jnp.float32),
        pltpu.VMEM((TOK_PER_W,), jnp.float32),
        pltpu.SemaphoreType.DMA,
    ],
    mesh=plsc.VectorSubcoreMesh(core_axis_name="c", subcore_axis_name="s"),
    compiler_params=pltpu.CompilerParams(needs_layout_passes=False),
)
def _pos_mass_sc(*args):
    _sc_body(*args)


def kernel(byte_ids, charge_table, position_table, mass_table):
    assert byte_ids.shape == (B, L)
    ids_flat = byte_ids.reshape(N_TOK).astype(jnp.int32)
    ids_blk = byte_ids.reshape(NW, 1, TC_BLK).astype(jnp.int32)
    pe = _pe_table()
    charge_f = _charge_tc(ids_blk, charge_table.astype(jnp.bfloat16), pe)
    pos_f, mass_f = _pos_mass_sc(
        ids_flat, position_table.reshape(256 * POS_DIM), mass_table.reshape(256))
    return (
        charge_f.reshape(B, L, D_MODEL),
        jnp.transpose(pos_f.reshape(B, POS_DIM, L), (0, 2, 1)),
        mass_f.reshape(B, L, 1),
    )


# speedup vs baseline: 5.2186x; 1.2339x over previous
import functools
import math

import jax
import jax.numpy as jnp
import numpy as np
from jax import lax
from jax.experimental import pallas as pl
from jax.experimental.pallas import tpu as pltpu
from jax.experimental.pallas import tpu_sc as plsc

D_MODEL = 1024
POS_DIM = 16
B, L = 4, 4096
N_TOK = B * L
NC, NS, LANES = 2, 16, 16
NW = NC * NS
TOK_PER_W = N_TOK // NW
L_PER_W = L // NW

TC_BLK = 1024
N_LBLK = L // TC_BLK


def _pe_table():
    position = np.arange(L)[:, None].astype(np.float32)
    div_term = np.exp(
        np.arange(0, D_MODEL, 2).astype(np.float32) * (-math.log(10000.0) / D_MODEL)
    )
    pe = np.zeros((L, D_MODEL), dtype=np.float32)
    pe[:, 0::2] = np.sin(position * div_term)
    pe[:, 1::2] = np.cos(position * div_term)
    return jnp.asarray(pe)


def _charge_tc_body(ids_ref, tab_ref, pe_ref, out_ref):
    ids = ids_ref[0, 0, :]
    onehot = (ids[:, None] == lax.broadcasted_iota(jnp.int32, (TC_BLK, 256), 1))
    onehot = onehot.astype(jnp.float32)
    rows = jax.lax.dot_general(
        onehot, tab_ref[...],
        dimension_numbers=(((1,), (0,)), ((), ())),
        preferred_element_type=jnp.float32)
    out_ref[...] = rows + pe_ref[...]


def _charge_tc(ids_blk, charge_table, pe):
    return pl.pallas_call(
        _charge_tc_body,
        grid=(N_LBLK, B),
        in_specs=[
            pl.BlockSpec((1, 1, TC_BLK), lambda l, b: (b * N_LBLK + l, 0, 0)),
            pl.BlockSpec((256, D_MODEL), lambda l, b: (0, 0)),
            pl.BlockSpec((TC_BLK, D_MODEL), lambda l, b: (l, 0)),
        ],
        out_specs=pl.BlockSpec((TC_BLK, D_MODEL), lambda l, b: (b * N_LBLK + l, 0)),
        out_shape=jax.ShapeDtypeStruct((N_TOK, D_MODEL), jnp.float32),
    )(ids_blk, charge_table, pe)


def _sc_body(ids_hbm, pos_hbm, mass_hbm,
             pos_out, mass_out,
             idx_v, ptab_v, pos_v, mtab_v, mass_v):
    wid = lax.axis_index("s") * NC + lax.axis_index("c")
    w128 = wid * L_PER_W

    for b in range(B):
        pltpu.sync_copy(ids_hbm.at[pl.ds(b * L + w128, L_PER_W)],
                        idx_v.at[pl.ds(b * L_PER_W, L_PER_W)])
    pltpu.sync_copy(pos_hbm, ptab_v)
    pltpu.sync_copy(mass_hbm, mtab_v)

    def _pos_step(j, carry):
        ids16 = idx_v[pl.ds(j * LANES, LANES)]
        flat_base = ids16 * POS_DIM
        row0 = (j // 8) * POS_DIM
        col = (j % 8) * LANES
        for c in range(POS_DIM):
            vals = plsc.load_gather(ptab_v, [flat_base + c])
            pos_v[row0 + c, pl.ds(col, LANES)] = vals
        return carry

    lax.fori_loop(0, TOK_PER_W // LANES, _pos_step, 0)
    for b in range(B):
        pltpu.sync_copy(
            pos_v.at[pl.ds(b * POS_DIM, POS_DIM)],
            pos_out.at[pl.ds(b * POS_DIM, POS_DIM), pl.ds(w128, L_PER_W)])

    def _mass_step(j, carry):
        ids16 = idx_v[pl.ds(j * LANES, LANES)]
        x = plsc.load_gather(mtab_v, [ids16])
        mass_v[pl.ds(j * LANES, LANES)] = 1.0 / (1.0 + jnp.exp(-x))
        return carry

    lax.fori_loop(0, TOK_PER_W // LANES, _mass_step, 0)
    for b in range(B):
        pltpu.sync_copy(mass_v.at[pl.ds(b * L_PER_W, L_PER_W)],
                        mass_out.at[pl.ds(b * L + w128, L_PER_W)])


@functools.partial(
    pl.kernel,
    out_type=[
        jax.ShapeDtypeStruct((B * POS_DIM, L), jnp.float32),
        jax.ShapeDtypeStruct((N_TOK,), jnp.float32),
    ],
    scratch_types=[
        pltpu.VMEM((TOK_PER_W,), jnp.int32),
        pltpu.VMEM((256 * POS_DIM,), jnp.float32),
        pltpu.VMEM((B * POS_DIM, L_PER_W), jnp.float32),
        pltpu.VMEM((256,), jnp.float32),
        pltpu.VMEM((TOK_PER_W,), jnp.float32),
    ],
    mesh=plsc.VectorSubcoreMesh(core_axis_name="c", subcore_axis_name="s"),
    compiler_params=pltpu.CompilerParams(needs_layout_passes=False),
)
def _pos_mass_sc(*args):
    _sc_body(*args)


def kernel(byte_ids, charge_table, position_table, mass_table):
    assert byte_ids.shape == (B, L)
    ids_flat = byte_ids.reshape(N_TOK).astype(jnp.int32)
    ids_blk = byte_ids.reshape(B * N_LBLK, 1, TC_BLK).astype(jnp.int32)
    pe = _pe_table()
    charge_f = _charge_tc(ids_blk, charge_table, pe)
    pos_f, mass_f = _pos_mass_sc(
        ids_flat, position_table.reshape(256 * POS_DIM), mass_table.reshape(256))
    return (
        charge_f.reshape(B, L, D_MODEL),
        jnp.transpose(pos_f.reshape(B, POS_DIM, L), (0, 2, 1)),
        mass_f.reshape(B, L, 1),
    )


# speedup vs baseline: 5.9955x; 1.1489x over previous
import functools
import math

import jax
import jax.numpy as jnp
import numpy as np
from jax import lax
from jax.experimental import pallas as pl
from jax.experimental.pallas import tpu as pltpu
from jax.experimental.pallas import tpu_sc as plsc

D_MODEL = 1024
POS_DIM = 16
B, L = 4, 4096
N_TOK = B * L
NC, NS, LANES = 2, 16, 16
NW = NC * NS
TOK_PER_W = N_TOK // NW
L_PER_W = L // NW

TC_BLK = 2048
N_LBLK = L // TC_BLK


def _pe_table():
    position = np.arange(L)[:, None].astype(np.float32)
    div_term = np.exp(
        np.arange(0, D_MODEL, 2).astype(np.float32) * (-math.log(10000.0) / D_MODEL)
    )
    pe = np.zeros((L, D_MODEL), dtype=np.float32)
    pe[:, 0::2] = np.sin(position * div_term)
    pe[:, 1::2] = np.cos(position * div_term)
    return jnp.asarray(pe.astype(jnp.bfloat16))


def _charge_tc_body(ids_ref, tab_ref, pe_ref, out_ref):
    l = pl.program_id(0)
    b = pl.program_id(1)
    ids = ids_ref[b, pl.ds(l * TC_BLK, TC_BLK)]
    onehot = (ids[:, None] == lax.broadcasted_iota(jnp.int32, (TC_BLK, 256), 1))
    onehot = onehot.astype(jnp.float32)
    rows = jax.lax.dot_general(
        onehot, tab_ref[...],
        dimension_numbers=(((1,), (0,)), ((), ())),
        preferred_element_type=jnp.float32)
    out_ref[...] = rows + pe_ref[...].astype(jnp.float32)


def _charge_tc(ids, charge_table, pe):
    return pl.pallas_call(
        _charge_tc_body,
        grid=(N_LBLK, B),
        in_specs=[
            pl.BlockSpec((B, L), lambda l, b: (0, 0)),
            pl.BlockSpec((256, D_MODEL), lambda l, b: (0, 0)),
            pl.BlockSpec((TC_BLK, D_MODEL), lambda l, b: (l, 0)),
        ],
        out_specs=pl.BlockSpec((TC_BLK, D_MODEL), lambda l, b: (b * N_LBLK + l, 0)),
        out_shape=jax.ShapeDtypeStruct((N_TOK, D_MODEL), jnp.float32),
    )(ids, charge_table, pe)


def _sc_body(ids_hbm, pos_hbm, mass_hbm,
             pos_out, mass_out,
             idx_v, ptab_v, pos_v, mtab_v, mass_v):
    wid = lax.axis_index("s") * NC + lax.axis_index("c")
    w128 = wid * L_PER_W

    for b in range(B):
        pltpu.sync_copy(ids_hbm.at[pl.ds(b * L + w128, L_PER_W)],
                        idx_v.at[pl.ds(b * L_PER_W, L_PER_W)])
    pltpu.sync_copy(pos_hbm, ptab_v)
    pltpu.sync_copy(mass_hbm, mtab_v)

    def _pos_step(j, carry):
        ids16 = idx_v[pl.ds(j * LANES, LANES)]
        flat_base = ids16 * POS_DIM
        row0 = (j // 8) * POS_DIM
        col = (j % 8) * LANES
        for c in range(POS_DIM):
            vals = plsc.load_gather(ptab_v, [flat_base + c])
            pos_v[row0 + c, pl.ds(col, LANES)] = vals
        return carry

    lax.fori_loop(0, TOK_PER_W // LANES, _pos_step, 0)
    for b in range(B):
        pltpu.sync_copy(
            pos_v.at[pl.ds(b * POS_DIM, POS_DIM)],
            pos_out.at[pl.ds(b * POS_DIM, POS_DIM), pl.ds(w128, L_PER_W)])

    def _mass_step(j, carry):
        ids16 = idx_v[pl.ds(j * LANES, LANES)]
        x = plsc.load_gather(mtab_v, [ids16])
        mass_v[pl.ds(j * LANES, LANES)] = 1.0 / (1.0 + jnp.exp(-x))
        return carry

    lax.fori_loop(0, TOK_PER_W // LANES, _mass_step, 0)
    for b in range(B):
        pltpu.sync_copy(mass_v.at[pl.ds(b * L_PER_W, L_PER_W)],
                        mass_out.at[pl.ds(b * L + w128, L_PER_W)])


@functools.partial(
    pl.kernel,
    out_type=[
        jax.ShapeDtypeStruct((B * POS_DIM, L), jnp.float32),
        jax.ShapeDtypeStruct((N_TOK,), jnp.float32),
    ],
    scratch_types=[
        pltpu.VMEM((TOK_PER_W,), jnp.int32),
        pltpu.VMEM((256 * POS_DIM,), jnp.float32),
        pltpu.VMEM((B * POS_DIM, L_PER_W), jnp.float32),
        pltpu.VMEM((256,), jnp.float32),
        pltpu.VMEM((TOK_PER_W,), jnp.float32),
    ],
    mesh=plsc.VectorSubcoreMesh(core_axis_name="c", subcore_axis_name="s"),
    compiler_params=pltpu.CompilerParams(needs_layout_passes=False),
)
def _pos_mass_sc(*args):
    _sc_body(*args)


def kernel(byte_ids, charge_table, position_table, mass_table):
    assert byte_ids.shape == (B, L)
    ids_flat = byte_ids.reshape(N_TOK).astype(jnp.int32)
    pe = _pe_table()
    charge_f = _charge_tc(byte_ids, charge_table, pe)
    pos_f, mass_f = _pos_mass_sc(
        ids_flat, position_table.reshape(256 * POS_DIM), mass_table.reshape(256))
    return (
        charge_f.reshape(B, L, D_MODEL),
        jnp.transpose(pos_f.reshape(B, POS_DIM, L), (0, 2, 1)),
        mass_f.reshape(B, L, 1),
    )


# speedup vs baseline: 6.3485x; 1.0589x over previous
import functools
import math

import jax
import jax.numpy as jnp
import numpy as np
from jax import lax
from jax.experimental import pallas as pl
from jax.experimental.pallas import tpu as pltpu
from jax.experimental.pallas import tpu_sc as plsc

D_MODEL = 1024
POS_DIM = 16
B, L = 4, 4096
N_TOK = B * L
NC, NS, LANES = 2, 16, 16
NW = NC * NS
TOK_PER_W = N_TOK // NW
L_PER_W = L // NW

TC_BLK = 4096
N_LBLK = L // TC_BLK


def _pe_table():
    position = np.arange(L)[:, None].astype(np.float32)
    div_term = np.exp(
        np.arange(0, D_MODEL, 2).astype(np.float32) * (-math.log(10000.0) / D_MODEL)
    )
    pe = np.zeros((L, D_MODEL), dtype=np.float32)
    pe[:, 0::2] = np.sin(position * div_term)
    pe[:, 1::2] = np.cos(position * div_term)
    return jnp.asarray(pe.astype(jnp.bfloat16))


def _charge_tc_body(ids_ref, tab_ref, pe_ref, out_ref):
    l = pl.program_id(0)
    b = pl.program_id(1)
    ids = ids_ref[b, pl.ds(l * TC_BLK, TC_BLK)]
    onehot = (ids[:, None] == lax.broadcasted_iota(jnp.int32, (TC_BLK, 256), 1))
    onehot = onehot.astype(jnp.float32)
    rows = jax.lax.dot_general(
        onehot, tab_ref[...],
        dimension_numbers=(((1,), (0,)), ((), ())),
        preferred_element_type=jnp.float32)
    out_ref[...] = rows + pe_ref[...].astype(jnp.float32)


def _charge_tc(ids, charge_table, pe):
    return pl.pallas_call(
        _charge_tc_body,
        grid=(N_LBLK, B),
        in_specs=[
            pl.BlockSpec((B, L), lambda l, b: (0, 0)),
            pl.BlockSpec((256, D_MODEL), lambda l, b: (0, 0)),
            pl.BlockSpec((TC_BLK, D_MODEL), lambda l, b: (l, 0)),
        ],
        out_specs=pl.BlockSpec((TC_BLK, D_MODEL), lambda l, b: (b * N_LBLK + l, 0)),
        out_shape=jax.ShapeDtypeStruct((N_TOK, D_MODEL), jnp.float32),
    )(ids, charge_table, pe)


def _sc_body(ids_hbm, pos_hbm, mass_hbm,
             pos_out, mass_out,
             idx_v, ptab_v, pos_v, mtab_v, mass_v):
    wid = lax.axis_index("s") * NC + lax.axis_index("c")
    w128 = wid * L_PER_W

    for b in range(B):
        pltpu.sync_copy(ids_hbm.at[pl.ds(b * L + w128, L_PER_W)],
                        idx_v.at[pl.ds(b * L_PER_W, L_PER_W)])
    pltpu.sync_copy(pos_hbm, ptab_v)
    pltpu.sync_copy(mass_hbm, mtab_v)

    def _pos_step(j, carry):
        ids16 = idx_v[pl.ds(j * LANES, LANES)]
        flat_base = ids16 * POS_DIM
        row0 = (j // 8) * POS_DIM
        col = (j % 8) * LANES
        for c in range(POS_DIM):
            vals = plsc.load_gather(ptab_v, [flat_base + c])
            pos_v[row0 + c, pl.ds(col, LANES)] = vals
        return carry

    lax.fori_loop(0, TOK_PER_W // LANES, _pos_step, 0)
    for b in range(B):
        pltpu.sync_copy(
            pos_v.at[pl.ds(b * POS_DIM, POS_DIM)],
            pos_out.at[pl.ds(b * POS_DIM, POS_DIM), pl.ds(w128, L_PER_W)])

    def _mass_step(j, carry):
        ids16 = idx_v[pl.ds(j * LANES, LANES)]
        x = plsc.load_gather(mtab_v, [ids16])
        mass_v[pl.ds(j * LANES, LANES)] = 1.0 / (1.0 + jnp.exp(-x))
        return carry

    lax.fori_loop(0, TOK_PER_W // LANES, _mass_step, 0)
    for b in range(B):
        pltpu.sync_copy(mass_v.at[pl.ds(b * L_PER_W, L_PER_W)],
                        mass_out.at[pl.ds(b * L + w128, L_PER_W)])


@functools.partial(
    pl.kernel,
    out_type=[
        jax.ShapeDtypeStruct((B * POS_DIM, L), jnp.float32),
        jax.ShapeDtypeStruct((N_TOK,), jnp.float32),
    ],
    scratch_types=[
        pltpu.VMEM((TOK_PER_W,), jnp.int32),
        pltpu.VMEM((256 * POS_DIM,), jnp.float32),
        pltpu.VMEM((B * POS_DIM, L_PER_W), jnp.float32),
        pltpu.VMEM((256,), jnp.float32),
        pltpu.VMEM((TOK_PER_W,), jnp.float32),
    ],
    mesh=plsc.VectorSubcoreMesh(core_axis_name="c", subcore_axis_name="s"),
    compiler_params=pltpu.CompilerParams(needs_layout_passes=False),
)
def _pos_mass_sc(*args):
    _sc_body(*args)


def kernel(byte_ids, charge_table, position_table, mass_table):
    assert byte_ids.shape == (B, L)
    ids_flat = byte_ids.reshape(N_TOK).astype(jnp.int32)
    pe = _pe_table()
    charge_f = _charge_tc(byte_ids, charge_table, pe)
    pos_f, mass_f = _pos_mass_sc(
        ids_flat, position_table.reshape(256 * POS_DIM), mass_table.reshape(256))
    return (
        charge_f.reshape(B, L, D_MODEL),
        jnp.transpose(pos_f.reshape(B, POS_DIM, L), (0, 2, 1)),
        mass_f.reshape(B, L, 1),
    )
